# baseline (speedup 1.0000x reference)
import jax
import jax.numpy as jnp
from jax.experimental import pallas as pl
from jax.experimental.pallas import tpu as pltpu


def _select_kernel(ab_ref, img_ref, mask_ref, boost_ref, out_ref):
    a = ab_ref[0]
    b = ab_ref[1]
    m = mask_ref[...]
    bt = boost_ref[...] * a + b
    out_ref[...] = jnp.where(m[:, None, :, :] < 0.5, bt, img_ref[...])


def kernel(standardization, batchimg, batchmask, boost):
    batchimg = batchimg.astype(jnp.float32)
    batchmask = batchmask.astype(jnp.float32)
    boost = boost.astype(jnp.float32)
    B, C, H, W = batchimg.shape
    std = jnp.asarray(standardization)
    a = jnp.where(std != 0, jnp.float32(1.0 / 3.9), jnp.float32(0.5 / 3.9))
    b = jnp.where(std != 0, jnp.float32(0.0), jnp.float32(0.5))
    ab = jnp.stack([a, b]).astype(jnp.float32)

    R = 128
    grid = (B, H // R)
    out = pl.pallas_call(
        _select_kernel,
        grid=grid,
        compiler_params=pltpu.CompilerParams(
            dimension_semantics=("parallel", "parallel"),
        ),
        in_specs=[
            pl.BlockSpec(memory_space=pltpu.SMEM),
            pl.BlockSpec((1, C, R, W), lambda i, j: (i, 0, j, 0)),
            pl.BlockSpec((1, R, W), lambda i, j: (i, j, 0)),
            pl.BlockSpec((1, C, R, W), lambda i, j: (i, 0, j, 0)),
        ],
        out_specs=pl.BlockSpec((1, C, R, W), lambda i, j: (i, 0, j, 0)),
        out_shape=jax.ShapeDtypeStruct((B, C, H, W), jnp.float32),
    )(ab, batchimg, batchmask, boost)
    return out


# speedup vs baseline: 1.3485x; 1.3485x over previous
import jax
import jax.numpy as jnp
from jax.experimental import pallas as pl
from jax.experimental.pallas import tpu as pltpu


def _select_kernel(ab_ref, img_ref, mask_ref, boost_ref, out_ref):
    a = ab_ref[0]
    b = ab_ref[1]
    m = mask_ref[...]
    bt = boost_ref[...] * a + b
    out_ref[...] = jnp.where(m[:, None, :, :] < 0.5, bt, img_ref[...])


def kernel(standardization, batchimg, batchmask, boost):
    batchimg = batchimg.astype(jnp.float32)
    batchmask = batchmask.astype(jnp.float32)
    boost = boost.astype(jnp.float32)
    B, C, H, W = batchimg.shape
    std = jnp.asarray(standardization)
    a = jnp.where(std != 0, jnp.float32(1.0 / 3.9), jnp.float32(0.5 / 3.9))
    b = jnp.where(std != 0, jnp.float32(0.0), jnp.float32(0.5))
    ab = jnp.stack([a, b]).astype(jnp.float32)

    R = 512
    grid = (B, H // R)
    out = pl.pallas_call(
        _select_kernel,
        grid=grid,
        compiler_params=pltpu.CompilerParams(
            dimension_semantics=("parallel", "parallel"),
        ),
        in_specs=[
            pl.BlockSpec(memory_space=pltpu.SMEM),
            pl.BlockSpec((1, C, R, W), lambda i, j: (i, 0, j, 0)),
            pl.BlockSpec((1, R, W), lambda i, j: (i, j, 0)),
            pl.BlockSpec((1, C, R, W), lambda i, j: (i, 0, j, 0)),
        ],
        out_specs=pl.BlockSpec((1, C, R, W), lambda i, j: (i, 0, j, 0)),
        out_shape=jax.ShapeDtypeStruct((B, C, H, W), jnp.float32),
    )(ab, batchimg, batchmask, boost)
    return out
